# baseline (speedup 1.0000x reference)
import jax
import jax.numpy as jnp
from jax.experimental import pallas as pl
from jax.experimental.pallas import tpu as pltpu

_NUM_EMB = 8192
_DIM = 32
_TOKENS = 8192
_BLK = 128
_NBLK = _TOKENS // _BLK
_CCOST = 0.25


def _vq_body(f_ref, emb_ref, enc_ref, zq_ref, loss_ref, ppl_ref,
             cnt_ref, acc_ref):
    i = pl.program_id(0)
    f = f_ref[...]
    emb = emb_ref[...]
    fn = jnp.sum(f * f, axis=1, keepdims=True)
    en = jnp.sum(emb * emb, axis=1)
    m = jnp.dot(f.astype(jnp.bfloat16), emb.astype(jnp.bfloat16).T,
                preferred_element_type=jnp.float32)
    d = (fn + en[None, :]) - 2.0 * m
    idx = jnp.argmin(d, axis=1)
    onehot = (jax.lax.broadcasted_iota(jnp.int32, (_BLK, _NUM_EMB), 1)
              == idx[:, None]).astype(jnp.float32)
    enc_ref[...] = onehot
    zq = jnp.dot(onehot, emb, preferred_element_type=jnp.float32)
    diff = zq - f
    zq_ref[...] = f + diff

    @pl.when(i == 0)
    def _init():
        cnt_ref[...] = jnp.zeros_like(cnt_ref)
        acc_ref[0] = 0.0

    cnt_ref[...] += jnp.sum(onehot, axis=0, keepdims=True)
    acc_ref[0] += jnp.sum(diff * diff)

    @pl.when(i == _NBLK - 1)
    def _fin():
        loss_ref[...] = jnp.reshape(_CCOST * (acc_ref[0] / (_TOKENS * _DIM)),
                                    (1, 1))
        p = cnt_ref[...] / _TOKENS
        ppl_ref[...] = jnp.reshape(jnp.exp(-jnp.sum(p * jnp.log(p + 1e-10))),
                                   (1, 1))


_vq_call = pl.pallas_call(
    _vq_body,
    grid=(_NBLK,),
    in_specs=[
        pl.BlockSpec((_BLK, _DIM), lambda i: (i, 0)),
        pl.BlockSpec((_NUM_EMB, _DIM), lambda i: (0, 0)),
    ],
    out_specs=[
        pl.BlockSpec((_BLK, _NUM_EMB), lambda i: (i, 0)),
        pl.BlockSpec((_BLK, _DIM), lambda i: (i, 0)),
        pl.BlockSpec((1, 1), lambda i: (0, 0)),
        pl.BlockSpec((1, 1), lambda i: (0, 0)),
    ],
    out_shape=[
        jax.ShapeDtypeStruct((_TOKENS, _NUM_EMB), jnp.float32),
        jax.ShapeDtypeStruct((_TOKENS, _DIM), jnp.float32),
        jax.ShapeDtypeStruct((1, 1), jnp.float32),
        jax.ShapeDtypeStruct((1, 1), jnp.float32),
    ],
    scratch_shapes=[
        pltpu.VMEM((1, _NUM_EMB), jnp.float32),
        pltpu.SMEM((1,), jnp.float32),
    ],
)


def kernel(z_e, emb_weight):
    b, dim, h, w = z_e.shape
    z = jnp.transpose(z_e, (0, 2, 3, 1))
    flat = z.reshape(-1, dim)
    enc, zq_st, loss, ppl = _vq_call(flat, emb_weight)
    z_q_out = jnp.transpose(zq_st.reshape(b, h, w, dim), (0, 3, 1, 2))
    return z_q_out, loss[0, 0], ppl[0, 0], enc


# speedup vs baseline: 1.1303x; 1.1303x over previous
import jax
import jax.numpy as jnp
from jax.experimental import pallas as pl
from jax.experimental.pallas import tpu as pltpu

_NUM_EMB = 8192
_DIM = 32
_TOKENS = 8192
_BLK = 512
_NBLK = _TOKENS // _BLK
_CCOST = 0.25


def _vq_body(f_ref, emb_ref, enc_ref, zq_ref, loss_ref, ppl_ref,
             cnt_ref, acc_ref):
    i = pl.program_id(0)
    f = f_ref[...]
    emb = emb_ref[...]
    fn = jnp.sum(f * f, axis=1, keepdims=True)
    en = jnp.sum(emb * emb, axis=1)
    m = jnp.dot(f.astype(jnp.bfloat16), emb.astype(jnp.bfloat16).T,
                preferred_element_type=jnp.float32)
    d = (fn + en[None, :]) - 2.0 * m
    idx = jnp.argmin(d, axis=1)
    onehot = (jax.lax.broadcasted_iota(jnp.int32, (_BLK, _NUM_EMB), 1)
              == idx[:, None]).astype(jnp.float32)
    enc_ref[...] = onehot
    zq = jnp.dot(onehot, emb, preferred_element_type=jnp.float32)
    diff = zq - f
    zq_ref[...] = f + diff

    @pl.when(i == 0)
    def _init():
        cnt_ref[...] = jnp.zeros_like(cnt_ref)
        acc_ref[0] = 0.0

    cnt_ref[...] += jnp.sum(onehot, axis=0, keepdims=True)
    acc_ref[0] += jnp.sum(diff * diff)

    @pl.when(i == _NBLK - 1)
    def _fin():
        loss_ref[...] = jnp.reshape(_CCOST * (acc_ref[0] / (_TOKENS * _DIM)),
                                    (1, 1))
        p = cnt_ref[...] / _TOKENS
        ppl_ref[...] = jnp.reshape(jnp.exp(-jnp.sum(p * jnp.log(p + 1e-10))),
                                   (1, 1))


_vq_call = pl.pallas_call(
    _vq_body,
    grid=(_NBLK,),
    in_specs=[
        pl.BlockSpec((_BLK, _DIM), lambda i: (i, 0)),
        pl.BlockSpec((_NUM_EMB, _DIM), lambda i: (0, 0)),
    ],
    out_specs=[
        pl.BlockSpec((_BLK, _NUM_EMB), lambda i: (i, 0)),
        pl.BlockSpec((_BLK, _DIM), lambda i: (i, 0)),
        pl.BlockSpec((1, 1), lambda i: (0, 0)),
        pl.BlockSpec((1, 1), lambda i: (0, 0)),
    ],
    out_shape=[
        jax.ShapeDtypeStruct((_TOKENS, _NUM_EMB), jnp.float32),
        jax.ShapeDtypeStruct((_TOKENS, _DIM), jnp.float32),
        jax.ShapeDtypeStruct((1, 1), jnp.float32),
        jax.ShapeDtypeStruct((1, 1), jnp.float32),
    ],
    scratch_shapes=[
        pltpu.VMEM((1, _NUM_EMB), jnp.float32),
        pltpu.SMEM((1,), jnp.float32),
    ],
)


def kernel(z_e, emb_weight):
    b, dim, h, w = z_e.shape
    z = jnp.transpose(z_e, (0, 2, 3, 1))
    flat = z.reshape(-1, dim)
    enc, zq_st, loss, ppl = _vq_call(flat, emb_weight)
    z_q_out = jnp.transpose(zq_st.reshape(b, h, w, dim), (0, 3, 1, 2))
    return z_q_out, loss[0, 0], ppl[0, 0], enc
